# Initial kernel scaffold; baseline (speedup 1.0000x reference)
import jax
import jax.numpy as jnp
from jax.experimental import pallas as pl


def kernel(x, edge_index, edge_attr, batch, W_l, b_l, W_r, b_r, W_e, att, conv_bias, pool_w):
    raise NotImplementedError("write your pallas kernel here")


# speedup vs baseline: 6.9795x; 6.9795x over previous
import functools

import jax
import jax.numpy as jnp
from jax import lax
from jax.experimental import pallas as pl
from jax.experimental.pallas import tpu as pltpu
from jax.experimental.pallas import tpu_sc as plsc

N = 10000
E = 320000
D_IN = 128
D_EDGE = 16
H = 8
C = 128
HC = H * C
G = 16
RATIO = 0.8

NC = 2
NS = 16
NW = NC * NS
CH = 80

NB = 80
NBLK = N // NB
EB = 512
EBLK = E // EB


def _sc_gather(table, idx, D):
    B = idx.shape[0]
    per_w = B // NW
    n_chunks = per_w // CH
    mesh = plsc.VectorSubcoreMesh(core_axis_name="c", subcore_axis_name="s")

    @functools.partial(
        pl.kernel, mesh=mesh,
        out_type=jax.ShapeDtypeStruct((B, D), jnp.float32),
        scratch_types=[
            pltpu.VMEM((CH,), jnp.int32),
            pltpu.VMEM((CH, D), jnp.float32),
            pltpu.SemaphoreType.DMA,
        ],
    )
    def k(table_hbm, idx_hbm, out_hbm, idx_v, rows_v, sem):
        wid = lax.axis_index("s") * NC + lax.axis_index("c")

        def body(j, carry):
            base = wid * per_w + j * CH
            pltpu.sync_copy(idx_hbm.at[pl.ds(base, CH)], idx_v)
            pltpu.async_copy(table_hbm.at[idx_v], rows_v, sem).wait()
            pltpu.sync_copy(rows_v, out_hbm.at[pl.ds(base, CH)])
            return carry

        lax.fori_loop(0, n_chunks, body, 0)

    return k(table, idx)


def _sc_scatter_add(vals, idx, zeros, D):
    B = idx.shape[0]
    per_w = B // NW
    n_chunks = per_w // CH
    stripe = 632
    mesh = plsc.VectorSubcoreMesh(core_axis_name="c", subcore_axis_name="s")

    @functools.partial(
        pl.kernel, mesh=mesh,
        out_type=jax.ShapeDtypeStruct((NC, N, D), jnp.float32),
        scratch_types=[
            pltpu.VMEM((CH,), jnp.int32),
            pltpu.VMEM((CH, D), jnp.float32),
            pltpu.VMEM_SHARED((N, D), jnp.float32),
        ],
    )
    def k(vals_hbm, idx_hbm, zeros_hbm, out_hbm, idx_v, vals_v, acc):
        cid = lax.axis_index("c")
        sid = lax.axis_index("s")
        wid = sid * NC + cid
        off = jnp.minimum(sid * stripe, N - stripe)

        pltpu.sync_copy(zeros_hbm.at[pl.ds(off, stripe)],
                        acc.at[pl.ds(off, stripe)])
        plsc.subcore_barrier()

        def body(j, carry):
            base = wid * per_w + j * CH
            pltpu.sync_copy(idx_hbm.at[pl.ds(base, CH)], idx_v)
            pltpu.sync_copy(vals_hbm.at[pl.ds(base, CH)], vals_v)
            pltpu.sync_copy(vals_v, acc.at[idx_v], add=True)
            return carry

        lax.fori_loop(0, n_chunks, body, 0)
        plsc.subcore_barrier()
        pltpu.sync_copy(acc.at[pl.ds(off, stripe)],
                        out_hbm.at[cid, pl.ds(off, stripe)])

    return k(vals, idx, zeros)


def _sc_scatter_msg(msg, idx, zeros):
    per_w = E // NW
    n_chunks = per_w // CH
    stripe = 632
    mesh = plsc.VectorSubcoreMesh(core_axis_name="c", subcore_axis_name="s")

    @functools.partial(
        pl.kernel, mesh=mesh,
        out_type=jax.ShapeDtypeStruct((H, NC, N, C), jnp.float32),
        scratch_types=[
            pltpu.VMEM((CH,), jnp.int32),
            pltpu.VMEM((CH, C), jnp.float32),
            pltpu.VMEM_SHARED((N, C), jnp.float32),
        ],
    )
    def k(msg_hbm, idx_hbm, zeros_hbm, out_hbm, idx_v, vals_v, acc):
        cid = lax.axis_index("c")
        sid = lax.axis_index("s")
        wid = sid * NC + cid
        off = jnp.minimum(sid * stripe, N - stripe)

        for h in range(H):
            pltpu.sync_copy(zeros_hbm.at[pl.ds(off, stripe)],
                            acc.at[pl.ds(off, stripe)])
            plsc.subcore_barrier()

            def body(j, carry):
                base = wid * per_w + j * CH
                pltpu.sync_copy(idx_hbm.at[pl.ds(base, CH)], idx_v)
                pltpu.sync_copy(msg_hbm.at[h, pl.ds(base, CH)], vals_v)
                pltpu.sync_copy(vals_v, acc.at[idx_v], add=True)
                return carry

            lax.fori_loop(0, n_chunks, body, 0)
            plsc.subcore_barrier()
            pltpu.sync_copy(acc.at[pl.ds(off, stripe)],
                            out_hbm.at[h, cid, pl.ds(off, stripe)])
            plsc.subcore_barrier()

    return k(msg, idx, zeros)


def _tc1_proj(x, W_l, b_l2, W_r, b_r2):
    def body(x_ref, wl_ref, bl_ref, wr_ref, br_ref, xl_ref, xr_ref):
        xb = x_ref[...]
        xl_ref[...] = jnp.dot(xb, wl_ref[...],
                              preferred_element_type=jnp.float32) + bl_ref[...]
        xr_ref[...] = jnp.dot(xb, wr_ref[...],
                              preferred_element_type=jnp.float32) + br_ref[...]

    return pl.pallas_call(
        body,
        grid=(NBLK,),
        in_specs=[
            pl.BlockSpec((NB, D_IN), lambda i: (i, 0)),
            pl.BlockSpec((D_IN, HC), lambda i: (0, 0)),
            pl.BlockSpec((1, HC), lambda i: (0, 0)),
            pl.BlockSpec((D_IN, HC), lambda i: (0, 0)),
            pl.BlockSpec((1, HC), lambda i: (0, 0)),
        ],
        out_specs=[
            pl.BlockSpec((NB, HC), lambda i: (i, 0)),
            pl.BlockSpec((NB, HC), lambda i: (i, 0)),
        ],
        out_shape=[
            jax.ShapeDtypeStruct((N, HC), jnp.float32),
            jax.ShapeDtypeStruct((N, HC), jnp.float32),
        ],
    )(x, W_l, b_l2, W_r, b_r2)


def _edge_ex(gz, att_ref, ex_ref, rows, pad):
    m = jnp.where(gz >= 0, gz, 0.2 * gz)
    for h in range(H):
        al = jnp.sum(m[:, h * C:(h + 1) * C] * att_ref[h, :][None, :], axis=1,
                     keepdims=True)
        ex_ref[:, pl.ds(h, 1)] = jnp.exp(al)
    ex_ref[:, pl.ds(H, pad - H)] = jnp.zeros((rows, pad - H), jnp.float32)


def _tc2_edge_alpha(gxl, gxr, edge_attr, W_e, att):
    def body(gxl_ref, gxr_ref, ea_ref, we_ref, att_ref, ex_ref):
        z = gxl_ref[...] + gxr_ref[...] + jnp.dot(
            ea_ref[...], we_ref[...], preferred_element_type=jnp.float32)
        _edge_ex(z, att_ref, ex_ref, EB, C)

    return pl.pallas_call(
        body,
        grid=(EBLK,),
        in_specs=[
            pl.BlockSpec((EB, HC), lambda i: (i, 0)),
            pl.BlockSpec((EB, HC), lambda i: (i, 0)),
            pl.BlockSpec((EB, D_EDGE), lambda i: (i, 0)),
            pl.BlockSpec((D_EDGE, HC), lambda i: (0, 0)),
            pl.BlockSpec((H, C), lambda i: (0, 0)),
        ],
        out_specs=pl.BlockSpec((EB, C), lambda i: (i, 0)),
        out_shape=jax.ShapeDtypeStruct((E, C), jnp.float32),
    )(gxl, gxr, edge_attr, W_e, att)


def _tc2b_self_alpha(x_l, x_r, la_part, W_e, att):
    def body(xl_ref, xr_ref, lap_ref, we_ref, att_ref, exs_ref):
        la = lap_ref[0] + lap_ref[1]
        loop_attr = la[:, :D_EDGE] / jnp.maximum(la[:, D_EDGE:D_EDGE + 1], 1.0)
        z = xl_ref[...] + xr_ref[...] + jnp.dot(
            loop_attr, we_ref[...], preferred_element_type=jnp.float32)
        _edge_ex(z, att_ref, exs_ref, NB, 16)

    return pl.pallas_call(
        body,
        grid=(NBLK,),
        in_specs=[
            pl.BlockSpec((NB, HC), lambda i: (i, 0)),
            pl.BlockSpec((NB, HC), lambda i: (i, 0)),
            pl.BlockSpec((NC, NB, C), lambda i: (0, i, 0)),
            pl.BlockSpec((D_EDGE, HC), lambda i: (0, 0)),
            pl.BlockSpec((H, C), lambda i: (0, 0)),
        ],
        out_specs=pl.BlockSpec((NB, 16), lambda i: (i, 0)),
        out_shape=jax.ShapeDtypeStruct((N, 16), jnp.float32),
    )(x_l, x_r, la_part, W_e, att)


def _tc3_denom(den_part, ex_self):
    def body(dp_ref, exs_ref, den_ref):
        den = dp_ref[0, :, :H] + dp_ref[1, :, :H] + exs_ref[:, :H]
        den_ref[:, pl.ds(0, H)] = den
        den_ref[:, pl.ds(H, 8)] = jnp.ones((NB, 8), jnp.float32)

    return pl.pallas_call(
        body,
        grid=(NBLK,),
        in_specs=[
            pl.BlockSpec((NC, NB, C), lambda i: (0, i, 0)),
            pl.BlockSpec((NB, 16), lambda i: (i, 0)),
        ],
        out_specs=pl.BlockSpec((NB, 16), lambda i: (i, 0)),
        out_shape=jax.ShapeDtypeStruct((N, 16), jnp.float32),
    )(den_part, ex_self)


def _tc4_msg(gxl, ex):
    def body(gxl_ref, ex_ref, msg_ref):
        for h in range(H):
            msg_ref[h] = ex_ref[:, h:h + 1] * gxl_ref[:, h * C:(h + 1) * C]

    return pl.pallas_call(
        body,
        grid=(EBLK,),
        in_specs=[
            pl.BlockSpec((EB, HC), lambda i: (i, 0)),
            pl.BlockSpec((EB, C), lambda i: (i, 0)),
        ],
        out_specs=pl.BlockSpec((H, EB, C), lambda i: (0, i, 0)),
        out_shape=jax.ShapeDtypeStruct((H, E, C), jnp.float32),
    )(gxl, ex)


def _tc5a_combine(out_part, ex_self, denom_pad, x_l, conv_bias2, pool_w2):
    def body(op_ref, exs_ref, den_ref, xl_ref, cb_ref, pw_ref, h_ref, s_ref):
        for h in range(H):
            hcol = (op_ref[h, 0] + op_ref[h, 1]
                    + exs_ref[:, h:h + 1] * xl_ref[:, h * C:(h + 1) * C]
                    ) / den_ref[:, h:h + 1]
            h_ref[:, pl.ds(h * C, C)] = hcol
        hb = jnp.maximum(h_ref[...] + cb_ref[...], 0.0)
        h_ref[...] = hb
        pw = pw_ref[...]
        nrm = jnp.sqrt(jnp.sum(pw * pw))
        sc = jnp.tanh(jnp.dot(hb, pw, preferred_element_type=jnp.float32) / nrm)
        s_ref[...] = jnp.broadcast_to(sc, (NB, 8))

    return pl.pallas_call(
        body,
        grid=(NBLK,),
        in_specs=[
            pl.BlockSpec((H, NC, NB, C), lambda i: (0, 0, i, 0)),
            pl.BlockSpec((NB, 16), lambda i: (i, 0)),
            pl.BlockSpec((NB, 16), lambda i: (i, 0)),
            pl.BlockSpec((NB, HC), lambda i: (i, 0)),
            pl.BlockSpec((1, HC), lambda i: (0, 0)),
            pl.BlockSpec((HC, 1), lambda i: (0, 0)),
        ],
        out_specs=[
            pl.BlockSpec((NB, HC), lambda i: (i, 0)),
            pl.BlockSpec((NB, 8), lambda i: (i, 0)),
        ],
        out_shape=[
            jax.ShapeDtypeStruct((N, HC), jnp.float32),
            jax.ShapeDtypeStruct((N, 8), jnp.float32),
        ],
    )(out_part, ex_self, denom_pad, x_l, conv_bias2, pool_w2)


def _tc5b_pool(hfull, score, batchf, scoreT, batchT):
    def body(h_ref, s_ref, b_ref, sallT_ref, ballT_ref, out_ref,
             gmp_ref, gap_ref):
        i = pl.program_id(0)

        @pl.when(i == 0)
        def _init():
            gmp_ref[...] = jnp.full((G, HC), -jnp.inf, jnp.float32)
            gap_ref[...] = jnp.zeros((G, HC), jnp.float32)

        s_blk = s_ref[:, 0:1]
        b_blk = b_ref[:, 0:1]
        s_all = sallT_ref[0:1, :]
        b_all = ballT_ref[0:1, :]

        gidx = (i * NB
                + lax.broadcasted_iota(jnp.int32, (NB, 1), 0)).astype(jnp.float32)
        jidx = lax.broadcasted_iota(jnp.int32, (1, N), 1).astype(jnp.float32)
        same = (b_blk == b_all).astype(jnp.float32)
        higher = jnp.where(
            (s_all > s_blk) | ((s_all == s_blk) & (jidx < gidx)), 1.0, 0.0)
        rank = jnp.sum(same * higher, axis=1, keepdims=True)

        cnt = jnp.sum(same, axis=1, keepdims=True)
        kb = jnp.ceil(RATIO * cnt)
        keep = (rank < kb).astype(jnp.float32)

        xp = h_ref[...] * s_blk
        for g in range(G):
            mg = ((b_blk == float(g)).astype(jnp.float32) * keep) > 0.5
            contrib_max = jnp.max(jnp.where(mg, xp, -jnp.inf), axis=0,
                                  keepdims=True)
            contrib_sum = jnp.sum(jnp.where(mg, xp, 0.0), axis=0,
                                  keepdims=True)
            gmp_ref[pl.ds(g, 1), :] = jnp.maximum(gmp_ref[pl.ds(g, 1), :],
                                                  contrib_max)
            gap_ref[pl.ds(g, 1), :] = gap_ref[pl.ds(g, 1), :] + contrib_sum

        @pl.when(i == NBLK - 1)
        def _fin():
            grange = lax.broadcasted_iota(jnp.int32, (G, 1), 0).astype(jnp.float32)
            counts = jnp.sum((b_all == grange).astype(jnp.float32),
                             axis=1, keepdims=True)
            kf = jnp.ceil(RATIO * counts)
            out_ref[:, pl.ds(0, HC)] = gmp_ref[...]
            out_ref[:, pl.ds(HC, HC)] = gap_ref[...] / kf

    return pl.pallas_call(
        body,
        grid=(NBLK,),
        in_specs=[
            pl.BlockSpec((NB, HC), lambda i: (i, 0)),
            pl.BlockSpec((NB, 8), lambda i: (i, 0)),
            pl.BlockSpec((NB, 8), lambda i: (i, 0)),
            pl.BlockSpec((8, N), lambda i: (0, 0)),
            pl.BlockSpec((8, N), lambda i: (0, 0)),
        ],
        out_specs=pl.BlockSpec((G, 2 * HC), lambda i: (0, 0)),
        out_shape=jax.ShapeDtypeStruct((G, 2 * HC), jnp.float32),
        scratch_shapes=[
            pltpu.VMEM((G, HC), jnp.float32),
            pltpu.VMEM((G, HC), jnp.float32),
        ],
    )(hfull, score, batchf, scoreT, batchT)


def kernel(x, edge_index, edge_attr, batch, W_l, b_l, W_r, b_r, W_e, att,
           conv_bias, pool_w):
    src = edge_index[0]
    dst = edge_index[1]
    b_l2 = b_l.reshape(1, HC)
    b_r2 = b_r.reshape(1, HC)
    cb2 = conv_bias.reshape(1, HC)
    pw2 = pool_w.reshape(HC, 1)
    batchf = jnp.broadcast_to(batch.astype(jnp.float32)[:, None], (N, 8))

    ea_pad = jnp.concatenate(
        [edge_attr, jnp.ones((E, 1), jnp.float32),
         jnp.zeros((E, C - D_EDGE - 1), jnp.float32)], axis=1)
    zeros128 = jnp.zeros((N, C), jnp.float32)

    x_l, x_r = _tc1_proj(x, W_l, b_l2, W_r, b_r2)

    la_part = _sc_scatter_add(ea_pad, dst, zeros128, C)
    gxl = _sc_gather(x_l, src, HC)
    gxr = _sc_gather(x_r, dst, HC)

    ex = _tc2_edge_alpha(gxl, gxr, edge_attr, W_e, att)
    ex_self = _tc2b_self_alpha(x_l, x_r, la_part, W_e, att)

    den_part = _sc_scatter_add(ex, dst, zeros128, C)
    denom_pad = _tc3_denom(den_part, ex_self)

    msg = _tc4_msg(gxl, ex)

    out_part = _sc_scatter_msg(msg, dst, zeros128)

    hfull, score = _tc5a_combine(out_part, ex_self, denom_pad, x_l, cb2, pw2)
    scoreT = jnp.broadcast_to(score[:, 0].reshape(1, N), (8, N))
    batchT = jnp.broadcast_to(batch.astype(jnp.float32).reshape(1, N), (8, N))
    return _tc5b_pool(hfull, score, batchf, scoreT, batchT)


# speedup vs baseline: 8.1195x; 1.1633x over previous
import functools

import jax
import jax.numpy as jnp
from jax import lax
from jax.experimental import pallas as pl
from jax.experimental.pallas import tpu as pltpu
from jax.experimental.pallas import tpu_sc as plsc

N = 10000
E = 320000
D_IN = 128
D_EDGE = 16
H = 8
C = 128
HC = H * C
G = 16
RATIO = 0.8

NC = 2
NS = 16
NW = NC * NS
CH = 80
CHG = 40

NB = 80
NBLK = N // NB
EB = 512
EBLK = E // EB


def _sc_gather(table, idx3, D):
    n_chunks = idx3.shape[1]
    per_w = n_chunks * CHG
    B = NW * per_w
    mesh = plsc.VectorSubcoreMesh(core_axis_name="c", subcore_axis_name="s")

    @functools.partial(
        pl.kernel, mesh=mesh,
        out_type=jax.ShapeDtypeStruct((B, D), jnp.float32),
        scratch_types=[
            pltpu.VMEM((n_chunks, CHG), jnp.int32),
            pltpu.VMEM((CHG, D), jnp.float32),
            pltpu.VMEM((CHG, D), jnp.float32),
            pltpu.SemaphoreType.DMA,
            pltpu.SemaphoreType.DMA,
            pltpu.SemaphoreType.DMA,
            pltpu.SemaphoreType.DMA,
        ],
    )
    def k(table_hbm, idx3_hbm, out_hbm, idx2_v, rows0, rows1,
          g0, g1, w0, w1):
        wid = lax.axis_index("s") * NC + lax.axis_index("c")
        pltpu.sync_copy(idx3_hbm.at[wid], idx2_v)

        def pair(p, carry):
            j0 = 2 * p
            j1 = 2 * p + 1
            b0 = wid * per_w + j0 * CHG
            b1 = wid * per_w + j1 * CHG
            pltpu.async_copy(table_hbm.at[idx2_v.at[j0]], rows0, g0).wait()
            cw1 = pltpu.async_copy(table_hbm.at[idx2_v.at[j1]], rows1, g1)
            pltpu.async_copy(rows0, out_hbm.at[pl.ds(b0, CHG)], w0).wait()
            cw1.wait()
            pltpu.async_copy(rows1, out_hbm.at[pl.ds(b1, CHG)], w1).wait()
            return carry

        lax.fori_loop(0, n_chunks // 2, pair, 0)
        if n_chunks % 2:
            j = n_chunks - 1
            base = wid * per_w + j * CHG
            pltpu.async_copy(table_hbm.at[idx2_v.at[j]], rows0, g0).wait()
            pltpu.async_copy(rows0, out_hbm.at[pl.ds(base, CHG)], w0).wait()

    return k(table, idx3)


NBUF = 3


def _scatter_groups(vals_hbm_row, idx2_v, vals_bufs, fsem, ssem, acc,
                    per_w, wid):
    n_chunks = per_w // CH
    n_groups = n_chunks // NBUF
    tail = n_chunks % NBUF

    def do_group(g, nb):
        fills = []
        for b in range(nb):
            base = wid * per_w + (g * NBUF + b) * CH
            fills.append(pltpu.async_copy(
                vals_hbm_row.at[pl.ds(base, CH)], vals_bufs[b], fsem))
        for f in fills:
            f.wait()
        scats = []
        for b in range(nb):
            j = g * NBUF + b
            scats.append(pltpu.async_copy(
                vals_bufs[b], acc.at[idx2_v.at[j]], ssem, add=True))
        for s in scats:
            s.wait()

    def group(g, carry):
        do_group(g, NBUF)
        return carry

    lax.fori_loop(0, n_groups, group, 0)
    if tail:
        do_group(n_groups, tail)


def _sc_scatter_add(vals, idx3, zeros, D):
    per_w = idx3.shape[1] * CH
    n_chunks = idx3.shape[1]
    stripe = 632
    mesh = plsc.VectorSubcoreMesh(core_axis_name="c", subcore_axis_name="s")

    @functools.partial(
        pl.kernel, mesh=mesh,
        out_type=jax.ShapeDtypeStruct((NC, N, D), jnp.float32),
        scratch_types=[
            pltpu.VMEM((n_chunks, CH), jnp.int32),
        ] + [pltpu.VMEM((CH, D), jnp.float32) for _ in range(NBUF)] + [
            pltpu.SemaphoreType.DMA,
            pltpu.SemaphoreType.DMA,
            pltpu.VMEM_SHARED((N, D), jnp.float32),
        ],
    )
    def k(vals_hbm, idx3_hbm, zeros_hbm, out_hbm, idx2_v, *rest):
        vals_bufs = rest[:NBUF]
        fsem, ssem, acc = rest[NBUF], rest[NBUF + 1], rest[NBUF + 2]
        cid = lax.axis_index("c")
        sid = lax.axis_index("s")
        wid = sid * NC + cid
        off = jnp.minimum(sid * stripe, N - stripe)

        pltpu.sync_copy(idx3_hbm.at[wid], idx2_v)
        pltpu.sync_copy(zeros_hbm.at[pl.ds(off, stripe)],
                        acc.at[pl.ds(off, stripe)])
        plsc.subcore_barrier()
        _scatter_groups(vals_hbm, idx2_v, vals_bufs, fsem, ssem, acc,
                        per_w, wid)
        plsc.subcore_barrier()
        pltpu.sync_copy(acc.at[pl.ds(off, stripe)],
                        out_hbm.at[cid, pl.ds(off, stripe)])

    return k(vals, idx3, zeros)


def _sc_scatter_msg(msg, idx3, zeros):
    per_w = E // NW
    n_chunks = per_w // CH
    stripe = 632
    mesh = plsc.VectorSubcoreMesh(core_axis_name="c", subcore_axis_name="s")

    @functools.partial(
        pl.kernel, mesh=mesh,
        out_type=jax.ShapeDtypeStruct((H, NC, N, C), jnp.float32),
        scratch_types=[
            pltpu.VMEM((n_chunks, CH), jnp.int32),
        ] + [pltpu.VMEM((CH, C), jnp.float32) for _ in range(NBUF)] + [
            pltpu.SemaphoreType.DMA,
            pltpu.SemaphoreType.DMA,
            pltpu.VMEM_SHARED((N, C), jnp.float32),
        ],
    )
    def k(msg_hbm, idx3_hbm, zeros_hbm, out_hbm, idx2_v, *rest):
        vals_bufs = rest[:NBUF]
        fsem, ssem, acc = rest[NBUF], rest[NBUF + 1], rest[NBUF + 2]
        cid = lax.axis_index("c")
        sid = lax.axis_index("s")
        wid = sid * NC + cid
        off = jnp.minimum(sid * stripe, N - stripe)

        pltpu.sync_copy(idx3_hbm.at[wid], idx2_v)
        for h in range(H):
            pltpu.sync_copy(zeros_hbm.at[pl.ds(off, stripe)],
                            acc.at[pl.ds(off, stripe)])
            plsc.subcore_barrier()
            _scatter_groups(msg_hbm.at[h], idx2_v, vals_bufs, fsem, ssem,
                            acc, per_w, wid)
            plsc.subcore_barrier()
            pltpu.sync_copy(acc.at[pl.ds(off, stripe)],
                            out_hbm.at[h, cid, pl.ds(off, stripe)])
            plsc.subcore_barrier()

    return k(msg, idx3, zeros)


def _tc1_proj(x, W_l, b_l2, W_r, b_r2):
    def body(x_ref, wl_ref, bl_ref, wr_ref, br_ref, xl_ref, xr_ref):
        xb = x_ref[...]
        xl_ref[...] = jnp.dot(xb, wl_ref[...],
                              preferred_element_type=jnp.float32) + bl_ref[...]
        xr_ref[...] = jnp.dot(xb, wr_ref[...],
                              preferred_element_type=jnp.float32) + br_ref[...]

    return pl.pallas_call(
        body,
        grid=(NBLK,),
        in_specs=[
            pl.BlockSpec((NB, D_IN), lambda i: (i, 0)),
            pl.BlockSpec((D_IN, HC), lambda i: (0, 0)),
            pl.BlockSpec((1, HC), lambda i: (0, 0)),
            pl.BlockSpec((D_IN, HC), lambda i: (0, 0)),
            pl.BlockSpec((1, HC), lambda i: (0, 0)),
        ],
        out_specs=[
            pl.BlockSpec((NB, HC), lambda i: (i, 0)),
            pl.BlockSpec((NB, HC), lambda i: (i, 0)),
        ],
        out_shape=[
            jax.ShapeDtypeStruct((N, HC), jnp.float32),
            jax.ShapeDtypeStruct((N, HC), jnp.float32),
        ],
    )(x, W_l, b_l2, W_r, b_r2)


def _edge_ex(gz, att_ref, ex_ref, rows, pad):
    m = jnp.where(gz >= 0, gz, 0.2 * gz)
    for h in range(H):
        al = jnp.sum(m[:, h * C:(h + 1) * C] * att_ref[h, :][None, :], axis=1,
                     keepdims=True)
        ex_ref[:, pl.ds(h, 1)] = jnp.exp(al)
    ex_ref[:, pl.ds(H, pad - H)] = jnp.zeros((rows, pad - H), jnp.float32)


def _tc2_edge_alpha(gxl, gxr, edge_attr, W_e, att):
    def body(gxl_ref, gxr_ref, ea_ref, we_ref, att_ref, ex_ref):
        z = gxl_ref[...] + gxr_ref[...] + jnp.dot(
            ea_ref[...], we_ref[...], preferred_element_type=jnp.float32)
        _edge_ex(z, att_ref, ex_ref, EB, C)

    return pl.pallas_call(
        body,
        grid=(EBLK,),
        in_specs=[
            pl.BlockSpec((EB, HC), lambda i: (i, 0)),
            pl.BlockSpec((EB, HC), lambda i: (i, 0)),
            pl.BlockSpec((EB, D_EDGE), lambda i: (i, 0)),
            pl.BlockSpec((D_EDGE, HC), lambda i: (0, 0)),
            pl.BlockSpec((H, C), lambda i: (0, 0)),
        ],
        out_specs=pl.BlockSpec((EB, C), lambda i: (i, 0)),
        out_shape=jax.ShapeDtypeStruct((E, C), jnp.float32),
    )(gxl, gxr, edge_attr, W_e, att)


def _tc2b_self_alpha(x_l, x_r, la_part, W_e, att):
    def body(xl_ref, xr_ref, lap_ref, we_ref, att_ref, exs_ref):
        la = lap_ref[0] + lap_ref[1]
        loop_attr = la[:, :D_EDGE] / jnp.maximum(la[:, D_EDGE:D_EDGE + 1], 1.0)
        z = xl_ref[...] + xr_ref[...] + jnp.dot(
            loop_attr, we_ref[...], preferred_element_type=jnp.float32)
        _edge_ex(z, att_ref, exs_ref, NB, 16)

    return pl.pallas_call(
        body,
        grid=(NBLK,),
        in_specs=[
            pl.BlockSpec((NB, HC), lambda i: (i, 0)),
            pl.BlockSpec((NB, HC), lambda i: (i, 0)),
            pl.BlockSpec((NC, NB, C), lambda i: (0, i, 0)),
            pl.BlockSpec((D_EDGE, HC), lambda i: (0, 0)),
            pl.BlockSpec((H, C), lambda i: (0, 0)),
        ],
        out_specs=pl.BlockSpec((NB, 16), lambda i: (i, 0)),
        out_shape=jax.ShapeDtypeStruct((N, 16), jnp.float32),
    )(x_l, x_r, la_part, W_e, att)


def _tc3_denom(den_part, ex_self):
    def body(dp_ref, exs_ref, den_ref):
        den = dp_ref[0, :, :H] + dp_ref[1, :, :H] + exs_ref[:, :H]
        den_ref[:, pl.ds(0, H)] = den
        den_ref[:, pl.ds(H, 8)] = jnp.ones((NB, 8), jnp.float32)

    return pl.pallas_call(
        body,
        grid=(NBLK,),
        in_specs=[
            pl.BlockSpec((NC, NB, C), lambda i: (0, i, 0)),
            pl.BlockSpec((NB, 16), lambda i: (i, 0)),
        ],
        out_specs=pl.BlockSpec((NB, 16), lambda i: (i, 0)),
        out_shape=jax.ShapeDtypeStruct((N, 16), jnp.float32),
    )(den_part, ex_self)


def _tc4_msg(gxl, ex):
    def body(gxl_ref, ex_ref, msg_ref):
        for h in range(H):
            msg_ref[h] = ex_ref[:, h:h + 1] * gxl_ref[:, h * C:(h + 1) * C]

    return pl.pallas_call(
        body,
        grid=(EBLK,),
        in_specs=[
            pl.BlockSpec((EB, HC), lambda i: (i, 0)),
            pl.BlockSpec((EB, C), lambda i: (i, 0)),
        ],
        out_specs=pl.BlockSpec((H, EB, C), lambda i: (0, i, 0)),
        out_shape=jax.ShapeDtypeStruct((H, E, C), jnp.float32),
    )(gxl, ex)


def _tc5a_combine(out_part, ex_self, denom_pad, x_l, conv_bias2, pool_w2):
    def body(op_ref, exs_ref, den_ref, xl_ref, cb_ref, pw_ref, h_ref, s_ref):
        for h in range(H):
            hcol = (op_ref[h, 0] + op_ref[h, 1]
                    + exs_ref[:, h:h + 1] * xl_ref[:, h * C:(h + 1) * C]
                    ) / den_ref[:, h:h + 1]
            h_ref[:, pl.ds(h * C, C)] = hcol
        hb = jnp.maximum(h_ref[...] + cb_ref[...], 0.0)
        h_ref[...] = hb
        pw = pw_ref[...]
        nrm = jnp.sqrt(jnp.sum(pw * pw))
        sc = jnp.tanh(jnp.dot(hb, pw, preferred_element_type=jnp.float32) / nrm)
        s_ref[...] = jnp.broadcast_to(sc, (NB, 8))

    return pl.pallas_call(
        body,
        grid=(NBLK,),
        in_specs=[
            pl.BlockSpec((H, NC, NB, C), lambda i: (0, 0, i, 0)),
            pl.BlockSpec((NB, 16), lambda i: (i, 0)),
            pl.BlockSpec((NB, 16), lambda i: (i, 0)),
            pl.BlockSpec((NB, HC), lambda i: (i, 0)),
            pl.BlockSpec((1, HC), lambda i: (0, 0)),
            pl.BlockSpec((HC, 1), lambda i: (0, 0)),
        ],
        out_specs=[
            pl.BlockSpec((NB, HC), lambda i: (i, 0)),
            pl.BlockSpec((NB, 8), lambda i: (i, 0)),
        ],
        out_shape=[
            jax.ShapeDtypeStruct((N, HC), jnp.float32),
            jax.ShapeDtypeStruct((N, 8), jnp.float32),
        ],
    )(out_part, ex_self, denom_pad, x_l, conv_bias2, pool_w2)


def _tc5b_pool(hfull, score, batchf, scoreT, batchT):
    def body(h_ref, s_ref, b_ref, sallT_ref, ballT_ref, out_ref,
             gmp_ref, gap_ref):
        i = pl.program_id(0)

        @pl.when(i == 0)
        def _init():
            gmp_ref[...] = jnp.full((G, HC), -jnp.inf, jnp.float32)
            gap_ref[...] = jnp.zeros((G, HC), jnp.float32)

        s_blk = s_ref[:, 0:1]
        b_blk = b_ref[:, 0:1]
        s_all = sallT_ref[0:1, :]
        b_all = ballT_ref[0:1, :]

        gidx = (i * NB
                + lax.broadcasted_iota(jnp.int32, (NB, 1), 0)).astype(jnp.float32)
        jidx = lax.broadcasted_iota(jnp.int32, (1, N), 1).astype(jnp.float32)
        same = (b_blk == b_all).astype(jnp.float32)
        higher = jnp.where(
            (s_all > s_blk) | ((s_all == s_blk) & (jidx < gidx)), 1.0, 0.0)
        rank = jnp.sum(same * higher, axis=1, keepdims=True)

        cnt = jnp.sum(same, axis=1, keepdims=True)
        kb = jnp.ceil(RATIO * cnt)
        keep = (rank < kb).astype(jnp.float32)

        xp = h_ref[...] * s_blk
        for g in range(G):
            mg = ((b_blk == float(g)).astype(jnp.float32) * keep) > 0.5
            contrib_max = jnp.max(jnp.where(mg, xp, -jnp.inf), axis=0,
                                  keepdims=True)
            contrib_sum = jnp.sum(jnp.where(mg, xp, 0.0), axis=0,
                                  keepdims=True)
            gmp_ref[pl.ds(g, 1), :] = jnp.maximum(gmp_ref[pl.ds(g, 1), :],
                                                  contrib_max)
            gap_ref[pl.ds(g, 1), :] = gap_ref[pl.ds(g, 1), :] + contrib_sum

        @pl.when(i == NBLK - 1)
        def _fin():
            grange = lax.broadcasted_iota(jnp.int32, (G, 1), 0).astype(jnp.float32)
            counts = jnp.sum((b_all == grange).astype(jnp.float32),
                             axis=1, keepdims=True)
            kf = jnp.ceil(RATIO * counts)
            out_ref[:, pl.ds(0, HC)] = gmp_ref[...]
            out_ref[:, pl.ds(HC, HC)] = gap_ref[...] / kf

    return pl.pallas_call(
        body,
        grid=(NBLK,),
        in_specs=[
            pl.BlockSpec((NB, HC), lambda i: (i, 0)),
            pl.BlockSpec((NB, 8), lambda i: (i, 0)),
            pl.BlockSpec((NB, 8), lambda i: (i, 0)),
            pl.BlockSpec((8, N), lambda i: (0, 0)),
            pl.BlockSpec((8, N), lambda i: (0, 0)),
        ],
        out_specs=pl.BlockSpec((G, 2 * HC), lambda i: (0, 0)),
        out_shape=jax.ShapeDtypeStruct((G, 2 * HC), jnp.float32),
        scratch_shapes=[
            pltpu.VMEM((G, HC), jnp.float32),
            pltpu.VMEM((G, HC), jnp.float32),
        ],
    )(hfull, score, batchf, scoreT, batchT)


def kernel(x, edge_index, edge_attr, batch, W_l, b_l, W_r, b_r, W_e, att,
           conv_bias, pool_w):
    src = edge_index[0]
    dst = edge_index[1]
    b_l2 = b_l.reshape(1, HC)
    b_r2 = b_r.reshape(1, HC)
    cb2 = conv_bias.reshape(1, HC)
    pw2 = pool_w.reshape(HC, 1)
    batchf = jnp.broadcast_to(batch.astype(jnp.float32)[:, None], (N, 8))

    ea_pad = jnp.concatenate(
        [edge_attr, jnp.ones((E, 1), jnp.float32),
         jnp.zeros((E, C - D_EDGE - 1), jnp.float32)], axis=1)
    zeros128 = jnp.zeros((N, C), jnp.float32)

    src3g = src.reshape(NW, E // (NW * CHG), CHG)
    dst3g = dst.reshape(NW, E // (NW * CHG), CHG)
    dst3s = dst.reshape(NW, E // (NW * CH), CH)

    x_l, x_r = _tc1_proj(x, W_l, b_l2, W_r, b_r2)

    la_part = _sc_scatter_add(ea_pad, dst3s, zeros128, C)
    gxl = _sc_gather(x_l, src3g, HC)
    gxr = _sc_gather(x_r, dst3g, HC)

    ex = _tc2_edge_alpha(gxl, gxr, edge_attr, W_e, att)
    ex_self = _tc2b_self_alpha(x_l, x_r, la_part, W_e, att)

    den_part = _sc_scatter_add(ex, dst3s, zeros128, C)
    denom_pad = _tc3_denom(den_part, ex_self)

    msg = _tc4_msg(gxl, ex)

    out_part = _sc_scatter_msg(msg, dst3s, zeros128)

    hfull, score = _tc5a_combine(out_part, ex_self, denom_pad, x_l, cb2, pw2)
    scoreT = jnp.broadcast_to(score[:, 0].reshape(1, N), (8, N))
    batchT = jnp.broadcast_to(batch.astype(jnp.float32).reshape(1, N), (8, N))
    return _tc5b_pool(hfull, score, batchf, scoreT, batchT)


# speedup vs baseline: 8.8096x; 1.0850x over previous
import functools

import jax
import jax.numpy as jnp
from jax import lax
from jax.experimental import pallas as pl
from jax.experimental.pallas import tpu as pltpu
from jax.experimental.pallas import tpu_sc as plsc

N = 10000
E = 320000
D_IN = 128
D_EDGE = 16
H = 8
C = 128
HC = H * C
G = 16
RATIO = 0.8

NC = 2
NS = 16
NW = NC * NS
CH = 80
CHG = 40

NB = 80
NBLK = N // NB
EB = 512
EBLK = E // EB


def _sc_gather(table, idx3, D):
    n_chunks = idx3.shape[1]
    per_w = n_chunks * CHG
    B = NW * per_w
    mesh = plsc.VectorSubcoreMesh(core_axis_name="c", subcore_axis_name="s")

    @functools.partial(
        pl.kernel, mesh=mesh,
        out_type=jax.ShapeDtypeStruct((B, D), jnp.float32),
        scratch_types=[
            pltpu.VMEM((n_chunks, CHG), jnp.int32),
            pltpu.VMEM((CHG, D), jnp.float32),
            pltpu.VMEM((CHG, D), jnp.float32),
            pltpu.SemaphoreType.DMA,
            pltpu.SemaphoreType.DMA,
            pltpu.SemaphoreType.DMA,
            pltpu.SemaphoreType.DMA,
        ],
    )
    def k(table_hbm, idx3_hbm, out_hbm, idx2_v, rows0, rows1,
          g0, g1, w0, w1):
        wid = lax.axis_index("s") * NC + lax.axis_index("c")
        pltpu.sync_copy(idx3_hbm.at[wid], idx2_v)

        def pair(p, carry):
            j0 = 2 * p
            j1 = 2 * p + 1
            b0 = wid * per_w + j0 * CHG
            b1 = wid * per_w + j1 * CHG
            pltpu.async_copy(table_hbm.at[idx2_v.at[j0]], rows0, g0).wait()
            cw1 = pltpu.async_copy(table_hbm.at[idx2_v.at[j1]], rows1, g1)
            pltpu.async_copy(rows0, out_hbm.at[pl.ds(b0, CHG)], w0).wait()
            cw1.wait()
            pltpu.async_copy(rows1, out_hbm.at[pl.ds(b1, CHG)], w1).wait()
            return carry

        lax.fori_loop(0, n_chunks // 2, pair, 0)
        if n_chunks % 2:
            j = n_chunks - 1
            base = wid * per_w + j * CHG
            pltpu.async_copy(table_hbm.at[idx2_v.at[j]], rows0, g0).wait()
            pltpu.async_copy(rows0, out_hbm.at[pl.ds(base, CHG)], w0).wait()

    return k(table, idx3)


NBUF = 3


def _scatter_groups(vals_hbm_row, idx2_v, vals_bufs, fsem, ssem, acc,
                    per_w, wid):
    n_chunks = per_w // CH
    n_groups = n_chunks // NBUF
    tail = n_chunks % NBUF

    def do_group(g, nb):
        fills = []
        for b in range(nb):
            base = wid * per_w + (g * NBUF + b) * CH
            fills.append(pltpu.async_copy(
                vals_hbm_row.at[pl.ds(base, CH)], vals_bufs[b], fsem))
        for f in fills:
            f.wait()
        scats = []
        for b in range(nb):
            j = g * NBUF + b
            scats.append(pltpu.async_copy(
                vals_bufs[b], acc.at[idx2_v.at[j]], ssem, add=True))
        for s in scats:
            s.wait()

    def group(g, carry):
        do_group(g, NBUF)
        return carry

    lax.fori_loop(0, n_groups, group, 0)
    if tail:
        do_group(n_groups, tail)


def _sc_scatter_add(vals, idx3, zeros, D):
    per_w = idx3.shape[1] * CH
    n_chunks = idx3.shape[1]
    stripe = 632
    mesh = plsc.VectorSubcoreMesh(core_axis_name="c", subcore_axis_name="s")

    @functools.partial(
        pl.kernel, mesh=mesh,
        out_type=jax.ShapeDtypeStruct((NC, N, D), jnp.float32),
        scratch_types=[
            pltpu.VMEM((n_chunks, CH), jnp.int32),
        ] + [pltpu.VMEM((CH, D), jnp.float32) for _ in range(NBUF)] + [
            pltpu.SemaphoreType.DMA,
            pltpu.SemaphoreType.DMA,
            pltpu.VMEM_SHARED((N, D), jnp.float32),
        ],
    )
    def k(vals_hbm, idx3_hbm, zeros_hbm, out_hbm, idx2_v, *rest):
        vals_bufs = rest[:NBUF]
        fsem, ssem, acc = rest[NBUF], rest[NBUF + 1], rest[NBUF + 2]
        cid = lax.axis_index("c")
        sid = lax.axis_index("s")
        wid = sid * NC + cid
        off = jnp.minimum(sid * stripe, N - stripe)

        pltpu.sync_copy(idx3_hbm.at[wid], idx2_v)
        pltpu.sync_copy(zeros_hbm.at[pl.ds(off, stripe)],
                        acc.at[pl.ds(off, stripe)])
        plsc.subcore_barrier()
        _scatter_groups(vals_hbm, idx2_v, vals_bufs, fsem, ssem, acc,
                        per_w, wid)
        plsc.subcore_barrier()
        pltpu.sync_copy(acc.at[pl.ds(off, stripe)],
                        out_hbm.at[cid, pl.ds(off, stripe)])

    return k(vals, idx3, zeros)


def _sc_scatter_msg(msg, idx3, zeros):
    per_w = E // NW
    n_chunks = per_w // CH
    stripe = 632
    mesh = plsc.VectorSubcoreMesh(core_axis_name="c", subcore_axis_name="s")

    @functools.partial(
        pl.kernel, mesh=mesh,
        out_type=jax.ShapeDtypeStruct((H, NC, N, C), jnp.float32),
        scratch_types=[
            pltpu.VMEM((n_chunks, CH), jnp.int32),
        ] + [pltpu.VMEM((CH, C), jnp.float32) for _ in range(NBUF)] + [
            pltpu.SemaphoreType.DMA,
            pltpu.SemaphoreType.DMA,
            pltpu.VMEM_SHARED((N, C), jnp.float32),
        ],
    )
    def k(msg_hbm, idx3_hbm, zeros_hbm, out_hbm, idx2_v, *rest):
        vals_bufs = rest[:NBUF]
        fsem, ssem, acc = rest[NBUF], rest[NBUF + 1], rest[NBUF + 2]
        cid = lax.axis_index("c")
        sid = lax.axis_index("s")
        wid = sid * NC + cid
        off = jnp.minimum(sid * stripe, N - stripe)

        pltpu.sync_copy(idx3_hbm.at[wid], idx2_v)
        for h in range(H):
            pltpu.sync_copy(zeros_hbm.at[pl.ds(off, stripe)],
                            acc.at[pl.ds(off, stripe)])
            plsc.subcore_barrier()
            _scatter_groups(msg_hbm.at[h], idx2_v, vals_bufs, fsem, ssem,
                            acc, per_w, wid)
            plsc.subcore_barrier()
            pltpu.sync_copy(acc.at[pl.ds(off, stripe)],
                            out_hbm.at[h, cid, pl.ds(off, stripe)])
            plsc.subcore_barrier()

    return k(msg, idx3, zeros)


def _tc1_proj(x, W_l, b_l2, W_r, b_r2):
    def body(x_ref, wl_ref, bl_ref, wr_ref, br_ref, xl_ref, xr_ref):
        xb = x_ref[...]
        xl_ref[...] = jnp.dot(xb, wl_ref[...],
                              preferred_element_type=jnp.float32) + bl_ref[...]
        xr_ref[...] = jnp.dot(xb, wr_ref[...],
                              preferred_element_type=jnp.float32) + br_ref[...]

    return pl.pallas_call(
        body,
        grid=(NBLK,),
        in_specs=[
            pl.BlockSpec((NB, D_IN), lambda i: (i, 0)),
            pl.BlockSpec((D_IN, HC), lambda i: (0, 0)),
            pl.BlockSpec((1, HC), lambda i: (0, 0)),
            pl.BlockSpec((D_IN, HC), lambda i: (0, 0)),
            pl.BlockSpec((1, HC), lambda i: (0, 0)),
        ],
        out_specs=[
            pl.BlockSpec((NB, HC), lambda i: (i, 0)),
            pl.BlockSpec((NB, HC), lambda i: (i, 0)),
        ],
        out_shape=[
            jax.ShapeDtypeStruct((N, HC), jnp.float32),
            jax.ShapeDtypeStruct((N, HC), jnp.float32),
        ],
    )(x, W_l, b_l2, W_r, b_r2)


def _edge_ex(gz, att_ref, ex_ref, rows, pad):
    m = jnp.where(gz >= 0, gz, 0.2 * gz)
    for h in range(H):
        al = jnp.sum(m[:, h * C:(h + 1) * C] * att_ref[h, :][None, :], axis=1,
                     keepdims=True)
        ex_ref[:, pl.ds(h, 1)] = jnp.exp(al)
    ex_ref[:, pl.ds(H, pad - H)] = jnp.zeros((rows, pad - H), jnp.float32)


def _tc2_edge_alpha(gxl, gxr, edge_attr, W_e, att):
    def body(gxl_ref, gxr_ref, ea_ref, we_ref, att_ref, ex_ref, msg_ref):
        z = gxl_ref[...] + gxr_ref[...] + jnp.dot(
            ea_ref[...], we_ref[...], preferred_element_type=jnp.float32)
        _edge_ex(z, att_ref, ex_ref, EB, C)
        for h in range(H):
            msg_ref[h] = ex_ref[:, h:h + 1] * gxl_ref[:, h * C:(h + 1) * C]

    return pl.pallas_call(
        body,
        grid=(EBLK,),
        in_specs=[
            pl.BlockSpec((EB, HC), lambda i: (i, 0)),
            pl.BlockSpec((EB, HC), lambda i: (i, 0)),
            pl.BlockSpec((EB, D_EDGE), lambda i: (i, 0)),
            pl.BlockSpec((D_EDGE, HC), lambda i: (0, 0)),
            pl.BlockSpec((H, C), lambda i: (0, 0)),
        ],
        out_specs=[
            pl.BlockSpec((EB, C), lambda i: (i, 0)),
            pl.BlockSpec((H, EB, C), lambda i: (0, i, 0)),
        ],
        out_shape=[
            jax.ShapeDtypeStruct((E, C), jnp.float32),
            jax.ShapeDtypeStruct((H, E, C), jnp.float32),
        ],
    )(gxl, gxr, edge_attr, W_e, att)


def _tc2b_self_alpha(x_l, x_r, la_part, W_e, att):
    def body(xl_ref, xr_ref, lap_ref, we_ref, att_ref, exs_ref):
        la = lap_ref[0] + lap_ref[1]
        loop_attr = la[:, :D_EDGE] / jnp.maximum(la[:, D_EDGE:D_EDGE + 1], 1.0)
        z = xl_ref[...] + xr_ref[...] + jnp.dot(
            loop_attr, we_ref[...], preferred_element_type=jnp.float32)
        _edge_ex(z, att_ref, exs_ref, NB, 16)

    return pl.pallas_call(
        body,
        grid=(NBLK,),
        in_specs=[
            pl.BlockSpec((NB, HC), lambda i: (i, 0)),
            pl.BlockSpec((NB, HC), lambda i: (i, 0)),
            pl.BlockSpec((NC, NB, C), lambda i: (0, i, 0)),
            pl.BlockSpec((D_EDGE, HC), lambda i: (0, 0)),
            pl.BlockSpec((H, C), lambda i: (0, 0)),
        ],
        out_specs=pl.BlockSpec((NB, 16), lambda i: (i, 0)),
        out_shape=jax.ShapeDtypeStruct((N, 16), jnp.float32),
    )(x_l, x_r, la_part, W_e, att)


def _tc3_denom(den_part, ex_self):
    def body(dp_ref, exs_ref, den_ref):
        den = dp_ref[0, :, :H] + dp_ref[1, :, :H] + exs_ref[:, :H]
        den_ref[:, pl.ds(0, H)] = den
        den_ref[:, pl.ds(H, 8)] = jnp.ones((NB, 8), jnp.float32)

    return pl.pallas_call(
        body,
        grid=(NBLK,),
        in_specs=[
            pl.BlockSpec((NC, NB, C), lambda i: (0, i, 0)),
            pl.BlockSpec((NB, 16), lambda i: (i, 0)),
        ],
        out_specs=pl.BlockSpec((NB, 16), lambda i: (i, 0)),
        out_shape=jax.ShapeDtypeStruct((N, 16), jnp.float32),
    )(den_part, ex_self)


def _tc5a_combine(out_part, ex_self, denom_pad, x_l, conv_bias2, pool_w2):
    def body(op_ref, exs_ref, den_ref, xl_ref, cb_ref, pw_ref, h_ref, s_ref):
        for h in range(H):
            hcol = (op_ref[h, 0] + op_ref[h, 1]
                    + exs_ref[:, h:h + 1] * xl_ref[:, h * C:(h + 1) * C]
                    ) / den_ref[:, h:h + 1]
            h_ref[:, pl.ds(h * C, C)] = hcol
        hb = jnp.maximum(h_ref[...] + cb_ref[...], 0.0)
        h_ref[...] = hb
        pw = pw_ref[...]
        nrm = jnp.sqrt(jnp.sum(pw * pw))
        sc = jnp.tanh(jnp.dot(hb, pw, preferred_element_type=jnp.float32) / nrm)
        s_ref[...] = jnp.broadcast_to(sc, (NB, 8))

    return pl.pallas_call(
        body,
        grid=(NBLK,),
        in_specs=[
            pl.BlockSpec((H, NC, NB, C), lambda i: (0, 0, i, 0)),
            pl.BlockSpec((NB, 16), lambda i: (i, 0)),
            pl.BlockSpec((NB, 16), lambda i: (i, 0)),
            pl.BlockSpec((NB, HC), lambda i: (i, 0)),
            pl.BlockSpec((1, HC), lambda i: (0, 0)),
            pl.BlockSpec((HC, 1), lambda i: (0, 0)),
        ],
        out_specs=[
            pl.BlockSpec((NB, HC), lambda i: (i, 0)),
            pl.BlockSpec((NB, 8), lambda i: (i, 0)),
        ],
        out_shape=[
            jax.ShapeDtypeStruct((N, HC), jnp.float32),
            jax.ShapeDtypeStruct((N, 8), jnp.float32),
        ],
    )(out_part, ex_self, denom_pad, x_l, conv_bias2, pool_w2)


def _tc5b_pool(hfull, score, batchf, scoreT, batchT):
    def body(h_ref, s_ref, b_ref, sallT_ref, ballT_ref, out_ref,
             gmp_ref, gap_ref):
        i = pl.program_id(0)

        @pl.when(i == 0)
        def _init():
            gmp_ref[...] = jnp.full((G, HC), -jnp.inf, jnp.float32)
            gap_ref[...] = jnp.zeros((G, HC), jnp.float32)

        s_blk = s_ref[:, 0:1]
        b_blk = b_ref[:, 0:1]
        s_all = sallT_ref[0:1, :]
        b_all = ballT_ref[0:1, :]

        gidx = (i * NB
                + lax.broadcasted_iota(jnp.int32, (NB, 1), 0)).astype(jnp.float32)
        jidx = lax.broadcasted_iota(jnp.int32, (1, N), 1).astype(jnp.float32)
        same = (b_blk == b_all).astype(jnp.float32)
        higher = jnp.where(
            (s_all > s_blk) | ((s_all == s_blk) & (jidx < gidx)), 1.0, 0.0)
        rank = jnp.sum(same * higher, axis=1, keepdims=True)

        cnt = jnp.sum(same, axis=1, keepdims=True)
        kb = jnp.ceil(RATIO * cnt)
        keep = (rank < kb).astype(jnp.float32)

        xp = h_ref[...] * s_blk
        for g in range(G):
            mg = ((b_blk == float(g)).astype(jnp.float32) * keep) > 0.5
            contrib_max = jnp.max(jnp.where(mg, xp, -jnp.inf), axis=0,
                                  keepdims=True)
            contrib_sum = jnp.sum(jnp.where(mg, xp, 0.0), axis=0,
                                  keepdims=True)
            gmp_ref[pl.ds(g, 1), :] = jnp.maximum(gmp_ref[pl.ds(g, 1), :],
                                                  contrib_max)
            gap_ref[pl.ds(g, 1), :] = gap_ref[pl.ds(g, 1), :] + contrib_sum

        @pl.when(i == NBLK - 1)
        def _fin():
            grange = lax.broadcasted_iota(jnp.int32, (G, 1), 0).astype(jnp.float32)
            counts = jnp.sum((b_all == grange).astype(jnp.float32),
                             axis=1, keepdims=True)
            kf = jnp.ceil(RATIO * counts)
            out_ref[:, pl.ds(0, HC)] = gmp_ref[...]
            out_ref[:, pl.ds(HC, HC)] = gap_ref[...] / kf

    return pl.pallas_call(
        body,
        grid=(NBLK,),
        in_specs=[
            pl.BlockSpec((NB, HC), lambda i: (i, 0)),
            pl.BlockSpec((NB, 8), lambda i: (i, 0)),
            pl.BlockSpec((NB, 8), lambda i: (i, 0)),
            pl.BlockSpec((8, N), lambda i: (0, 0)),
            pl.BlockSpec((8, N), lambda i: (0, 0)),
        ],
        out_specs=pl.BlockSpec((G, 2 * HC), lambda i: (0, 0)),
        out_shape=jax.ShapeDtypeStruct((G, 2 * HC), jnp.float32),
        scratch_shapes=[
            pltpu.VMEM((G, HC), jnp.float32),
            pltpu.VMEM((G, HC), jnp.float32),
        ],
    )(hfull, score, batchf, scoreT, batchT)


def kernel(x, edge_index, edge_attr, batch, W_l, b_l, W_r, b_r, W_e, att,
           conv_bias, pool_w):
    src = edge_index[0]
    dst = edge_index[1]
    b_l2 = b_l.reshape(1, HC)
    b_r2 = b_r.reshape(1, HC)
    cb2 = conv_bias.reshape(1, HC)
    pw2 = pool_w.reshape(HC, 1)
    batchf = jnp.broadcast_to(batch.astype(jnp.float32)[:, None], (N, 8))

    ea_pad = jnp.concatenate(
        [edge_attr, jnp.ones((E, 1), jnp.float32),
         jnp.zeros((E, C - D_EDGE - 1), jnp.float32)], axis=1)
    zeros128 = jnp.zeros((N, C), jnp.float32)

    src3g = src.reshape(NW, E // (NW * CHG), CHG)
    dst3g = dst.reshape(NW, E // (NW * CHG), CHG)
    dst3s = dst.reshape(NW, E // (NW * CH), CH)

    x_l, x_r = _tc1_proj(x, W_l, b_l2, W_r, b_r2)

    la_part = _sc_scatter_add(ea_pad, dst3s, zeros128, C)
    gxl = _sc_gather(x_l, src3g, HC)
    gxr = _sc_gather(x_r, dst3g, HC)

    ex, msg = _tc2_edge_alpha(gxl, gxr, edge_attr, W_e, att)
    ex_self = _tc2b_self_alpha(x_l, x_r, la_part, W_e, att)

    den_part = _sc_scatter_add(ex, dst3s, zeros128, C)
    denom_pad = _tc3_denom(den_part, ex_self)

    out_part = _sc_scatter_msg(msg, dst3s, zeros128)

    hfull, score = _tc5a_combine(out_part, ex_self, denom_pad, x_l, cb2, pw2)
    scoreT = jnp.broadcast_to(score[:, 0].reshape(1, N), (8, N))
    batchT = jnp.broadcast_to(batch.astype(jnp.float32).reshape(1, N), (8, N))
    return _tc5b_pool(hfull, score, batchf, scoreT, batchT)


# speedup vs baseline: 8.9191x; 1.0124x over previous
import functools

import jax
import jax.numpy as jnp
from jax import lax
from jax.experimental import pallas as pl
from jax.experimental.pallas import tpu as pltpu
from jax.experimental.pallas import tpu_sc as plsc

N = 10000
E = 320000
D_IN = 128
D_EDGE = 16
H = 8
C = 128
HC = H * C
G = 16
RATIO = 0.8

NC = 2
NS = 16
NW = NC * NS
CH = 80
CHG = 40

NB = 80
NBLK = N // NB
EB = 512
EBLK = E // EB


def _sc_gather(table, idx3, D):
    n_chunks = idx3.shape[1]
    per_w = n_chunks * CHG
    B = NW * per_w
    n_pairs = n_chunks // 2
    assert n_chunks % 2 == 0
    mesh = plsc.VectorSubcoreMesh(core_axis_name="c", subcore_axis_name="s")

    @functools.partial(
        pl.kernel, mesh=mesh,
        out_type=jax.ShapeDtypeStruct((B, D), jnp.float32),
        scratch_types=[
            pltpu.VMEM((n_chunks, CHG), jnp.int32),
            pltpu.VMEM((CHG, D), jnp.float32),
            pltpu.VMEM((CHG, D), jnp.float32),
            pltpu.SemaphoreType.DMA,
            pltpu.SemaphoreType.DMA,
            pltpu.SemaphoreType.DMA,
            pltpu.SemaphoreType.DMA,
        ],
    )
    def k(table_hbm, idx3_hbm, out_hbm, idx2_v, rows0, rows1,
          g0, g1, w0, w1):
        wid = lax.axis_index("s") * NC + lax.axis_index("c")
        pltpu.sync_copy(idx3_hbm.at[wid], idx2_v)

        pltpu.async_copy(table_hbm.at[idx2_v.at[0]], rows0, g0)

        def pair(p, carry):
            j0 = 2 * p
            j1 = 2 * p + 1
            b0 = wid * per_w + j0 * CHG
            b1 = wid * per_w + j1 * CHG
            pltpu.make_async_copy(table_hbm.at[idx2_v.at[j0]], rows0,
                                  g0).wait()

            @pl.when(p > 0)
            def _drain_w1():
                pltpu.make_async_copy(rows1, out_hbm.at[pl.ds(b1, CHG)],
                                      w1).wait()

            pltpu.async_copy(table_hbm.at[idx2_v.at[j1]], rows1, g1)
            pltpu.async_copy(rows0, out_hbm.at[pl.ds(b0, CHG)], w0)

            @pl.when(p < n_pairs - 1)
            def _next_g0():
                pltpu.make_async_copy(rows0, out_hbm.at[pl.ds(b0, CHG)],
                                      w0).wait()
                pltpu.async_copy(table_hbm.at[idx2_v.at[j0 + 2]], rows0, g0)

            pltpu.make_async_copy(table_hbm.at[idx2_v.at[j1]], rows1,
                                  g1).wait()
            pltpu.async_copy(rows1, out_hbm.at[pl.ds(b1, CHG)], w1)
            return carry

        lax.fori_loop(0, n_pairs, pair, 0)
        last0 = wid * per_w + (n_chunks - 2) * CHG
        last1 = wid * per_w + (n_chunks - 1) * CHG
        pltpu.make_async_copy(rows0, out_hbm.at[pl.ds(last0, CHG)], w0).wait()
        pltpu.make_async_copy(rows1, out_hbm.at[pl.ds(last1, CHG)], w1).wait()

    return k(table, idx3)


NBUF = 3


def _scatter_groups(vals_hbm_row, idx2_v, vals_bufs, fsem, ssem, acc,
                    per_w, wid):
    n_chunks = per_w // CH
    n_groups = n_chunks // NBUF
    tail = n_chunks % NBUF

    def do_group(g, nb):
        fills = []
        for b in range(nb):
            base = wid * per_w + (g * NBUF + b) * CH
            fills.append(pltpu.async_copy(
                vals_hbm_row.at[pl.ds(base, CH)], vals_bufs[b], fsem))
        for f in fills:
            f.wait()
        scats = []
        for b in range(nb):
            j = g * NBUF + b
            scats.append(pltpu.async_copy(
                vals_bufs[b], acc.at[idx2_v.at[j]], ssem, add=True))
        for s in scats:
            s.wait()

    def group(g, carry):
        do_group(g, NBUF)
        return carry

    lax.fori_loop(0, n_groups, group, 0)
    if tail:
        do_group(n_groups, tail)


def _sc_scatter_add(vals, idx3, zeros, D):
    per_w = idx3.shape[1] * CH
    n_chunks = idx3.shape[1]
    stripe = 632
    mesh = plsc.VectorSubcoreMesh(core_axis_name="c", subcore_axis_name="s")

    @functools.partial(
        pl.kernel, mesh=mesh,
        out_type=jax.ShapeDtypeStruct((NC, N, D), jnp.float32),
        scratch_types=[
            pltpu.VMEM((n_chunks, CH), jnp.int32),
        ] + [pltpu.VMEM((CH, D), jnp.float32) for _ in range(NBUF)] + [
            pltpu.SemaphoreType.DMA,
            pltpu.SemaphoreType.DMA,
            pltpu.VMEM_SHARED((N, D), jnp.float32),
        ],
    )
    def k(vals_hbm, idx3_hbm, zeros_hbm, out_hbm, idx2_v, *rest):
        vals_bufs = rest[:NBUF]
        fsem, ssem, acc = rest[NBUF], rest[NBUF + 1], rest[NBUF + 2]
        cid = lax.axis_index("c")
        sid = lax.axis_index("s")
        wid = sid * NC + cid
        off = jnp.minimum(sid * stripe, N - stripe)

        pltpu.sync_copy(idx3_hbm.at[wid], idx2_v)
        pltpu.sync_copy(zeros_hbm.at[pl.ds(off, stripe)],
                        acc.at[pl.ds(off, stripe)])
        plsc.subcore_barrier()
        _scatter_groups(vals_hbm, idx2_v, vals_bufs, fsem, ssem, acc,
                        per_w, wid)
        plsc.subcore_barrier()
        pltpu.sync_copy(acc.at[pl.ds(off, stripe)],
                        out_hbm.at[cid, pl.ds(off, stripe)])

    return k(vals, idx3, zeros)


def _sc_scatter_msg(msg, idx3, zeros):
    per_w = E // NW
    n_chunks = per_w // CH
    stripe = 632
    mesh = plsc.VectorSubcoreMesh(core_axis_name="c", subcore_axis_name="s")

    @functools.partial(
        pl.kernel, mesh=mesh,
        out_type=jax.ShapeDtypeStruct((H, NC, N, C), jnp.float32),
        scratch_types=[
            pltpu.VMEM((n_chunks, CH), jnp.int32),
        ] + [pltpu.VMEM((CH, C), jnp.float32) for _ in range(NBUF)] + [
            pltpu.SemaphoreType.DMA,
            pltpu.SemaphoreType.DMA,
            pltpu.VMEM_SHARED((N, C), jnp.float32),
        ],
    )
    def k(msg_hbm, idx3_hbm, zeros_hbm, out_hbm, idx2_v, *rest):
        vals_bufs = rest[:NBUF]
        fsem, ssem, acc = rest[NBUF], rest[NBUF + 1], rest[NBUF + 2]
        cid = lax.axis_index("c")
        sid = lax.axis_index("s")
        wid = sid * NC + cid
        off = jnp.minimum(sid * stripe, N - stripe)

        pltpu.sync_copy(idx3_hbm.at[wid], idx2_v)
        for h in range(H):
            pltpu.sync_copy(zeros_hbm.at[pl.ds(off, stripe)],
                            acc.at[pl.ds(off, stripe)])
            plsc.subcore_barrier()
            _scatter_groups(msg_hbm.at[h], idx2_v, vals_bufs, fsem, ssem,
                            acc, per_w, wid)
            plsc.subcore_barrier()
            pltpu.sync_copy(acc.at[pl.ds(off, stripe)],
                            out_hbm.at[h, cid, pl.ds(off, stripe)])
            plsc.subcore_barrier()

    return k(msg, idx3, zeros)


def _tc1_proj(x, W_l, b_l2, W_r, b_r2):
    def body(x_ref, wl_ref, bl_ref, wr_ref, br_ref, xl_ref, xr_ref):
        xb = x_ref[...]
        xl_ref[...] = jnp.dot(xb, wl_ref[...],
                              preferred_element_type=jnp.float32) + bl_ref[...]
        xr_ref[...] = jnp.dot(xb, wr_ref[...],
                              preferred_element_type=jnp.float32) + br_ref[...]

    return pl.pallas_call(
        body,
        grid=(NBLK,),
        in_specs=[
            pl.BlockSpec((NB, D_IN), lambda i: (i, 0)),
            pl.BlockSpec((D_IN, HC), lambda i: (0, 0)),
            pl.BlockSpec((1, HC), lambda i: (0, 0)),
            pl.BlockSpec((D_IN, HC), lambda i: (0, 0)),
            pl.BlockSpec((1, HC), lambda i: (0, 0)),
        ],
        out_specs=[
            pl.BlockSpec((NB, HC), lambda i: (i, 0)),
            pl.BlockSpec((NB, HC), lambda i: (i, 0)),
        ],
        out_shape=[
            jax.ShapeDtypeStruct((N, HC), jnp.float32),
            jax.ShapeDtypeStruct((N, HC), jnp.float32),
        ],
    )(x, W_l, b_l2, W_r, b_r2)


def _edge_ex(gz, att_ref, ex_ref, rows, pad):
    m = jnp.where(gz >= 0, gz, 0.2 * gz)
    for h in range(H):
        al = jnp.sum(m[:, h * C:(h + 1) * C] * att_ref[h, :][None, :], axis=1,
                     keepdims=True)
        ex_ref[:, pl.ds(h, 1)] = jnp.exp(al)
    ex_ref[:, pl.ds(H, pad - H)] = jnp.zeros((rows, pad - H), jnp.float32)


def _tc2_edge_alpha(gxl, gxr, edge_attr, W_e, att):
    def body(gxl_ref, gxr_ref, ea_ref, we_ref, att_ref, ex_ref, msg_ref):
        z = gxl_ref[...] + gxr_ref[...] + jnp.dot(
            ea_ref[...], we_ref[...], preferred_element_type=jnp.float32)
        _edge_ex(z, att_ref, ex_ref, EB, C)
        for h in range(H):
            msg_ref[h] = ex_ref[:, h:h + 1] * gxl_ref[:, h * C:(h + 1) * C]

    return pl.pallas_call(
        body,
        grid=(EBLK,),
        in_specs=[
            pl.BlockSpec((EB, HC), lambda i: (i, 0)),
            pl.BlockSpec((EB, HC), lambda i: (i, 0)),
            pl.BlockSpec((EB, D_EDGE), lambda i: (i, 0)),
            pl.BlockSpec((D_EDGE, HC), lambda i: (0, 0)),
            pl.BlockSpec((H, C), lambda i: (0, 0)),
        ],
        out_specs=[
            pl.BlockSpec((EB, C), lambda i: (i, 0)),
            pl.BlockSpec((H, EB, C), lambda i: (0, i, 0)),
        ],
        out_shape=[
            jax.ShapeDtypeStruct((E, C), jnp.float32),
            jax.ShapeDtypeStruct((H, E, C), jnp.float32),
        ],
    )(gxl, gxr, edge_attr, W_e, att)


def _tc2b_self_alpha(x_l, x_r, la_part, W_e, att):
    def body(xl_ref, xr_ref, lap_ref, we_ref, att_ref, exs_ref):
        la = lap_ref[0] + lap_ref[1]
        loop_attr = la[:, :D_EDGE] / jnp.maximum(la[:, D_EDGE:D_EDGE + 1], 1.0)
        z = xl_ref[...] + xr_ref[...] + jnp.dot(
            loop_attr, we_ref[...], preferred_element_type=jnp.float32)
        _edge_ex(z, att_ref, exs_ref, NB, 16)

    return pl.pallas_call(
        body,
        grid=(NBLK,),
        in_specs=[
            pl.BlockSpec((NB, HC), lambda i: (i, 0)),
            pl.BlockSpec((NB, HC), lambda i: (i, 0)),
            pl.BlockSpec((NC, NB, C), lambda i: (0, i, 0)),
            pl.BlockSpec((D_EDGE, HC), lambda i: (0, 0)),
            pl.BlockSpec((H, C), lambda i: (0, 0)),
        ],
        out_specs=pl.BlockSpec((NB, 16), lambda i: (i, 0)),
        out_shape=jax.ShapeDtypeStruct((N, 16), jnp.float32),
    )(x_l, x_r, la_part, W_e, att)


def _tc3_denom(den_part, ex_self):
    def body(dp_ref, exs_ref, den_ref):
        den = dp_ref[0, :, :H] + dp_ref[1, :, :H] + exs_ref[:, :H]
        den_ref[:, pl.ds(0, H)] = den
        den_ref[:, pl.ds(H, 8)] = jnp.ones((NB, 8), jnp.float32)

    return pl.pallas_call(
        body,
        grid=(NBLK,),
        in_specs=[
            pl.BlockSpec((NC, NB, C), lambda i: (0, i, 0)),
            pl.BlockSpec((NB, 16), lambda i: (i, 0)),
        ],
        out_specs=pl.BlockSpec((NB, 16), lambda i: (i, 0)),
        out_shape=jax.ShapeDtypeStruct((N, 16), jnp.float32),
    )(den_part, ex_self)


def _tc5a_combine(out_part, ex_self, denom_pad, x_l, conv_bias2, pool_w2):
    def body(op_ref, exs_ref, den_ref, xl_ref, cb_ref, pw_ref, h_ref, s_ref):
        for h in range(H):
            hcol = (op_ref[h, 0] + op_ref[h, 1]
                    + exs_ref[:, h:h + 1] * xl_ref[:, h * C:(h + 1) * C]
                    ) / den_ref[:, h:h + 1]
            h_ref[:, pl.ds(h * C, C)] = hcol
        hb = jnp.maximum(h_ref[...] + cb_ref[...], 0.0)
        h_ref[...] = hb
        pw = pw_ref[...]
        nrm = jnp.sqrt(jnp.sum(pw * pw))
        sc = jnp.tanh(jnp.dot(hb, pw, preferred_element_type=jnp.float32) / nrm)
        s_ref[...] = jnp.broadcast_to(sc, (NB, 8))

    return pl.pallas_call(
        body,
        grid=(NBLK,),
        in_specs=[
            pl.BlockSpec((H, NC, NB, C), lambda i: (0, 0, i, 0)),
            pl.BlockSpec((NB, 16), lambda i: (i, 0)),
            pl.BlockSpec((NB, 16), lambda i: (i, 0)),
            pl.BlockSpec((NB, HC), lambda i: (i, 0)),
            pl.BlockSpec((1, HC), lambda i: (0, 0)),
            pl.BlockSpec((HC, 1), lambda i: (0, 0)),
        ],
        out_specs=[
            pl.BlockSpec((NB, HC), lambda i: (i, 0)),
            pl.BlockSpec((NB, 8), lambda i: (i, 0)),
        ],
        out_shape=[
            jax.ShapeDtypeStruct((N, HC), jnp.float32),
            jax.ShapeDtypeStruct((N, 8), jnp.float32),
        ],
    )(out_part, ex_self, denom_pad, x_l, conv_bias2, pool_w2)


def _tc5b_pool(hfull, score, batchf, scoreT, batchT):
    def body(h_ref, s_ref, b_ref, sallT_ref, ballT_ref, out_ref,
             gmp_ref, gap_ref):
        i = pl.program_id(0)

        @pl.when(i == 0)
        def _init():
            gmp_ref[...] = jnp.full((G, HC), -jnp.inf, jnp.float32)
            gap_ref[...] = jnp.zeros((G, HC), jnp.float32)

        s_blk = s_ref[:, 0:1]
        b_blk = b_ref[:, 0:1]
        s_all = sallT_ref[0:1, :]
        b_all = ballT_ref[0:1, :]

        gidx = (i * NB
                + lax.broadcasted_iota(jnp.int32, (NB, 1), 0)).astype(jnp.float32)
        jidx = lax.broadcasted_iota(jnp.int32, (1, N), 1).astype(jnp.float32)
        same = (b_blk == b_all).astype(jnp.float32)
        higher = jnp.where(
            (s_all > s_blk) | ((s_all == s_blk) & (jidx < gidx)), 1.0, 0.0)
        rank = jnp.sum(same * higher, axis=1, keepdims=True)

        cnt = jnp.sum(same, axis=1, keepdims=True)
        kb = jnp.ceil(RATIO * cnt)
        keep = (rank < kb).astype(jnp.float32)

        xp = h_ref[...] * s_blk
        for g in range(G):
            mg = ((b_blk == float(g)).astype(jnp.float32) * keep) > 0.5
            contrib_max = jnp.max(jnp.where(mg, xp, -jnp.inf), axis=0,
                                  keepdims=True)
            contrib_sum = jnp.sum(jnp.where(mg, xp, 0.0), axis=0,
                                  keepdims=True)
            gmp_ref[pl.ds(g, 1), :] = jnp.maximum(gmp_ref[pl.ds(g, 1), :],
                                                  contrib_max)
            gap_ref[pl.ds(g, 1), :] = gap_ref[pl.ds(g, 1), :] + contrib_sum

        @pl.when(i == NBLK - 1)
        def _fin():
            grange = lax.broadcasted_iota(jnp.int32, (G, 1), 0).astype(jnp.float32)
            counts = jnp.sum((b_all == grange).astype(jnp.float32),
                             axis=1, keepdims=True)
            kf = jnp.ceil(RATIO * counts)
            out_ref[:, pl.ds(0, HC)] = gmp_ref[...]
            out_ref[:, pl.ds(HC, HC)] = gap_ref[...] / kf

    return pl.pallas_call(
        body,
        grid=(NBLK,),
        in_specs=[
            pl.BlockSpec((NB, HC), lambda i: (i, 0)),
            pl.BlockSpec((NB, 8), lambda i: (i, 0)),
            pl.BlockSpec((NB, 8), lambda i: (i, 0)),
            pl.BlockSpec((8, N), lambda i: (0, 0)),
            pl.BlockSpec((8, N), lambda i: (0, 0)),
        ],
        out_specs=pl.BlockSpec((G, 2 * HC), lambda i: (0, 0)),
        out_shape=jax.ShapeDtypeStruct((G, 2 * HC), jnp.float32),
        scratch_shapes=[
            pltpu.VMEM((G, HC), jnp.float32),
            pltpu.VMEM((G, HC), jnp.float32),
        ],
    )(hfull, score, batchf, scoreT, batchT)


def kernel(x, edge_index, edge_attr, batch, W_l, b_l, W_r, b_r, W_e, att,
           conv_bias, pool_w):
    src = edge_index[0]
    dst = edge_index[1]
    b_l2 = b_l.reshape(1, HC)
    b_r2 = b_r.reshape(1, HC)
    cb2 = conv_bias.reshape(1, HC)
    pw2 = pool_w.reshape(HC, 1)
    batchf = jnp.broadcast_to(batch.astype(jnp.float32)[:, None], (N, 8))

    ea_pad = jnp.concatenate(
        [edge_attr, jnp.ones((E, 1), jnp.float32),
         jnp.zeros((E, C - D_EDGE - 1), jnp.float32)], axis=1)
    zeros128 = jnp.zeros((N, C), jnp.float32)

    src3g = src.reshape(NW, E // (NW * CHG), CHG)
    dst3g = dst.reshape(NW, E // (NW * CHG), CHG)
    dst3s = dst.reshape(NW, E // (NW * CH), CH)

    x_l, x_r = _tc1_proj(x, W_l, b_l2, W_r, b_r2)

    la_part = _sc_scatter_add(ea_pad, dst3s, zeros128, C)
    gxl = _sc_gather(x_l, src3g, HC)
    gxr = _sc_gather(x_r, dst3g, HC)

    ex, msg = _tc2_edge_alpha(gxl, gxr, edge_attr, W_e, att)
    ex_self = _tc2b_self_alpha(x_l, x_r, la_part, W_e, att)

    den_part = _sc_scatter_add(ex, dst3s, zeros128, C)
    denom_pad = _tc3_denom(den_part, ex_self)

    out_part = _sc_scatter_msg(msg, dst3s, zeros128)

    hfull, score = _tc5a_combine(out_part, ex_self, denom_pad, x_l, cb2, pw2)
    scoreT = jnp.broadcast_to(score[:, 0].reshape(1, N), (8, N))
    batchT = jnp.broadcast_to(batch.astype(jnp.float32).reshape(1, N), (8, N))
    return _tc5b_pool(hfull, score, batchf, scoreT, batchT)


# speedup vs baseline: 8.9427x; 1.0026x over previous
import functools

import jax
import jax.numpy as jnp
from jax import lax
from jax.experimental import pallas as pl
from jax.experimental.pallas import tpu as pltpu
from jax.experimental.pallas import tpu_sc as plsc

N = 10000
E = 320000
D_IN = 128
D_EDGE = 16
H = 8
C = 128
HC = H * C
G = 16
RATIO = 0.8

NC = 2
NS = 16
NW = NC * NS
CH = 80
CHG = 40

NB = 80
NBLK = N // NB
EB = 512
EBLK = E // EB


def _sc_gather2(table_l, table_r, idxpack, D):
    n_chunks = idxpack.shape[1]
    per_w = n_chunks * CHG
    B = NW * per_w
    mesh = plsc.VectorSubcoreMesh(core_axis_name="c", subcore_axis_name="s")

    @functools.partial(
        pl.kernel, mesh=mesh,
        out_type=[
            jax.ShapeDtypeStruct((B, D), jnp.float32),
            jax.ShapeDtypeStruct((B, D), jnp.float32),
        ],
        scratch_types=[
            pltpu.VMEM((n_chunks, 128), jnp.int32),
            pltpu.VMEM((CHG, D), jnp.float32),
            pltpu.VMEM((CHG, D), jnp.float32),
            pltpu.SemaphoreType.DMA,
            pltpu.SemaphoreType.DMA,
            pltpu.SemaphoreType.DMA,
            pltpu.SemaphoreType.DMA,
        ],
    )
    def k(tl_hbm, tr_hbm, ip_hbm, outl_hbm, outr_hbm,
          ip_v, rows0, rows1, g0, g1, w0, w1):
        wid = lax.axis_index("s") * NC + lax.axis_index("c")
        pltpu.sync_copy(ip_hbm.at[wid], ip_v)

        def il(p):
            return ip_v.at[p, pl.ds(0, CHG)]

        def ir(p):
            return ip_v.at[p, pl.ds(64, CHG)]

        pltpu.async_copy(tl_hbm.at[il(0)], rows0, g0)

        def step(p, carry):
            base = wid * per_w + p * CHG
            pltpu.make_async_copy(tl_hbm.at[il(p)], rows0, g0).wait()

            @pl.when(p > 0)
            def _drain_wr():
                pltpu.make_async_copy(rows1, outr_hbm.at[pl.ds(base, CHG)],
                                      w1).wait()

            pltpu.async_copy(tr_hbm.at[ir(p)], rows1, g1)
            pltpu.async_copy(rows0, outl_hbm.at[pl.ds(base, CHG)], w0)

            @pl.when(p < n_chunks - 1)
            def _next_gl():
                pltpu.make_async_copy(rows0, outl_hbm.at[pl.ds(base, CHG)],
                                      w0).wait()
                pltpu.async_copy(tl_hbm.at[il(p + 1)], rows0, g0)

            pltpu.make_async_copy(tr_hbm.at[ir(p)], rows1, g1).wait()
            pltpu.async_copy(rows1, outr_hbm.at[pl.ds(base, CHG)], w1)
            return carry

        lax.fori_loop(0, n_chunks, step, 0)
        last = wid * per_w + (n_chunks - 1) * CHG
        pltpu.make_async_copy(rows0, outl_hbm.at[pl.ds(last, CHG)], w0).wait()
        pltpu.make_async_copy(rows1, outr_hbm.at[pl.ds(last, CHG)], w1).wait()

    return k(table_l, table_r, idxpack)


NBUF = 3


def _scatter_groups(vals_hbm_row, idx2_v, vals_bufs, fsem, ssem, acc,
                    per_w, wid):
    n_chunks = per_w // CH
    n_groups = n_chunks // NBUF
    tail = n_chunks % NBUF

    def do_group(g, nb):
        fills = []
        for b in range(nb):
            base = wid * per_w + (g * NBUF + b) * CH
            fills.append(pltpu.async_copy(
                vals_hbm_row.at[pl.ds(base, CH)], vals_bufs[b], fsem))
        for f in fills:
            f.wait()
        scats = []
        for b in range(nb):
            j = g * NBUF + b
            scats.append(pltpu.async_copy(
                vals_bufs[b], acc.at[idx2_v.at[j]], ssem, add=True))
        for s in scats:
            s.wait()

    def group(g, carry):
        do_group(g, NBUF)
        return carry

    lax.fori_loop(0, n_groups, group, 0)
    if tail:
        do_group(n_groups, tail)


def _sc_scatter_add(vals, idx3, zeros, D):
    per_w = idx3.shape[1] * CH
    n_chunks = idx3.shape[1]
    stripe = 632
    mesh = plsc.VectorSubcoreMesh(core_axis_name="c", subcore_axis_name="s")

    @functools.partial(
        pl.kernel, mesh=mesh,
        out_type=jax.ShapeDtypeStruct((NC, N, D), jnp.float32),
        scratch_types=[
            pltpu.VMEM((n_chunks, CH), jnp.int32),
        ] + [pltpu.VMEM((CH, D), jnp.float32) for _ in range(NBUF)] + [
            pltpu.SemaphoreType.DMA,
            pltpu.SemaphoreType.DMA,
            pltpu.VMEM_SHARED((N, D), jnp.float32),
        ],
    )
    def k(vals_hbm, idx3_hbm, zeros_hbm, out_hbm, idx2_v, *rest):
        vals_bufs = rest[:NBUF]
        fsem, ssem, acc = rest[NBUF], rest[NBUF + 1], rest[NBUF + 2]
        cid = lax.axis_index("c")
        sid = lax.axis_index("s")
        wid = sid * NC + cid
        off = jnp.minimum(sid * stripe, N - stripe)

        pltpu.sync_copy(idx3_hbm.at[wid], idx2_v)
        pltpu.sync_copy(zeros_hbm.at[pl.ds(off, stripe)],
                        acc.at[pl.ds(off, stripe)])
        plsc.subcore_barrier()
        _scatter_groups(vals_hbm, idx2_v, vals_bufs, fsem, ssem, acc,
                        per_w, wid)
        plsc.subcore_barrier()
        pltpu.sync_copy(acc.at[pl.ds(off, stripe)],
                        out_hbm.at[cid, pl.ds(off, stripe)])

    return k(vals, idx3, zeros)


def _sc_scatter_msg(msg, idx3, zeros):
    per_w = E // NW
    n_chunks = per_w // CH
    stripe = 632
    mesh = plsc.VectorSubcoreMesh(core_axis_name="c", subcore_axis_name="s")

    @functools.partial(
        pl.kernel, mesh=mesh,
        out_type=jax.ShapeDtypeStruct((H, NC, N, C), jnp.float32),
        scratch_types=[
            pltpu.VMEM((n_chunks, CH), jnp.int32),
        ] + [pltpu.VMEM((CH, C), jnp.float32) for _ in range(NBUF)] + [
            pltpu.SemaphoreType.DMA,
            pltpu.SemaphoreType.DMA,
            pltpu.VMEM_SHARED((N, C), jnp.float32),
        ],
    )
    def k(msg_hbm, idx3_hbm, zeros_hbm, out_hbm, idx2_v, *rest):
        vals_bufs = rest[:NBUF]
        fsem, ssem, acc = rest[NBUF], rest[NBUF + 1], rest[NBUF + 2]
        cid = lax.axis_index("c")
        sid = lax.axis_index("s")
        wid = sid * NC + cid
        off = jnp.minimum(sid * stripe, N - stripe)

        pltpu.sync_copy(idx3_hbm.at[wid], idx2_v)
        for h in range(H):
            pltpu.sync_copy(zeros_hbm.at[pl.ds(off, stripe)],
                            acc.at[pl.ds(off, stripe)])
            plsc.subcore_barrier()
            _scatter_groups(msg_hbm.at[h], idx2_v, vals_bufs, fsem, ssem,
                            acc, per_w, wid)
            plsc.subcore_barrier()
            pltpu.sync_copy(acc.at[pl.ds(off, stripe)],
                            out_hbm.at[h, cid, pl.ds(off, stripe)])
            plsc.subcore_barrier()

    return k(msg, idx3, zeros)


def _tc1_proj(x, W_l, b_l2, W_r, b_r2):
    def body(x_ref, wl_ref, bl_ref, wr_ref, br_ref, xl_ref, xr_ref):
        xb = x_ref[...]
        xl_ref[...] = jnp.dot(xb, wl_ref[...],
                              preferred_element_type=jnp.float32) + bl_ref[...]
        xr_ref[...] = jnp.dot(xb, wr_ref[...],
                              preferred_element_type=jnp.float32) + br_ref[...]

    return pl.pallas_call(
        body,
        grid=(NBLK,),
        in_specs=[
            pl.BlockSpec((NB, D_IN), lambda i: (i, 0)),
            pl.BlockSpec((D_IN, HC), lambda i: (0, 0)),
            pl.BlockSpec((1, HC), lambda i: (0, 0)),
            pl.BlockSpec((D_IN, HC), lambda i: (0, 0)),
            pl.BlockSpec((1, HC), lambda i: (0, 0)),
        ],
        out_specs=[
            pl.BlockSpec((NB, HC), lambda i: (i, 0)),
            pl.BlockSpec((NB, HC), lambda i: (i, 0)),
        ],
        out_shape=[
            jax.ShapeDtypeStruct((N, HC), jnp.float32),
            jax.ShapeDtypeStruct((N, HC), jnp.float32),
        ],
    )(x, W_l, b_l2, W_r, b_r2)


def _edge_ex(gz, att_ref, ex_ref, rows, pad):
    m = jnp.where(gz >= 0, gz, 0.2 * gz)
    for h in range(H):
        al = jnp.sum(m[:, h * C:(h + 1) * C] * att_ref[h, :][None, :], axis=1,
                     keepdims=True)
        ex_ref[:, pl.ds(h, 1)] = jnp.exp(al)
    ex_ref[:, pl.ds(H, pad - H)] = jnp.zeros((rows, pad - H), jnp.float32)


def _tc2_edge_alpha(gxl, gxr, edge_attr, W_e, att):
    def body(gxl_ref, gxr_ref, ea_ref, we_ref, att_ref, ex_ref, msg_ref):
        z = gxl_ref[...] + gxr_ref[...] + jnp.dot(
            ea_ref[...], we_ref[...], preferred_element_type=jnp.float32)
        _edge_ex(z, att_ref, ex_ref, EB, C)
        for h in range(H):
            msg_ref[h] = ex_ref[:, h:h + 1] * gxl_ref[:, h * C:(h + 1) * C]

    return pl.pallas_call(
        body,
        grid=(EBLK,),
        in_specs=[
            pl.BlockSpec((EB, HC), lambda i: (i, 0)),
            pl.BlockSpec((EB, HC), lambda i: (i, 0)),
            pl.BlockSpec((EB, D_EDGE), lambda i: (i, 0)),
            pl.BlockSpec((D_EDGE, HC), lambda i: (0, 0)),
            pl.BlockSpec((H, C), lambda i: (0, 0)),
        ],
        out_specs=[
            pl.BlockSpec((EB, C), lambda i: (i, 0)),
            pl.BlockSpec((H, EB, C), lambda i: (0, i, 0)),
        ],
        out_shape=[
            jax.ShapeDtypeStruct((E, C), jnp.float32),
            jax.ShapeDtypeStruct((H, E, C), jnp.float32),
        ],
    )(gxl, gxr, edge_attr, W_e, att)


def _tc2b_self_alpha(x_l, x_r, la_part, W_e, att):
    def body(xl_ref, xr_ref, lap_ref, we_ref, att_ref, exs_ref):
        la = lap_ref[0] + lap_ref[1]
        loop_attr = la[:, :D_EDGE] / jnp.maximum(la[:, D_EDGE:D_EDGE + 1], 1.0)
        z = xl_ref[...] + xr_ref[...] + jnp.dot(
            loop_attr, we_ref[...], preferred_element_type=jnp.float32)
        _edge_ex(z, att_ref, exs_ref, NB, 16)

    return pl.pallas_call(
        body,
        grid=(NBLK,),
        in_specs=[
            pl.BlockSpec((NB, HC), lambda i: (i, 0)),
            pl.BlockSpec((NB, HC), lambda i: (i, 0)),
            pl.BlockSpec((NC, NB, C), lambda i: (0, i, 0)),
            pl.BlockSpec((D_EDGE, HC), lambda i: (0, 0)),
            pl.BlockSpec((H, C), lambda i: (0, 0)),
        ],
        out_specs=pl.BlockSpec((NB, 16), lambda i: (i, 0)),
        out_shape=jax.ShapeDtypeStruct((N, 16), jnp.float32),
    )(x_l, x_r, la_part, W_e, att)


def _tc3_denom(den_part, ex_self):
    def body(dp_ref, exs_ref, den_ref):
        den = dp_ref[0, :, :H] + dp_ref[1, :, :H] + exs_ref[:, :H]
        den_ref[:, pl.ds(0, H)] = den
        den_ref[:, pl.ds(H, 8)] = jnp.ones((NB, 8), jnp.float32)

    return pl.pallas_call(
        body,
        grid=(NBLK,),
        in_specs=[
            pl.BlockSpec((NC, NB, C), lambda i: (0, i, 0)),
            pl.BlockSpec((NB, 16), lambda i: (i, 0)),
        ],
        out_specs=pl.BlockSpec((NB, 16), lambda i: (i, 0)),
        out_shape=jax.ShapeDtypeStruct((N, 16), jnp.float32),
    )(den_part, ex_self)


def _tc5a_combine(out_part, ex_self, denom_pad, x_l, conv_bias2, pool_w2):
    def body(op_ref, exs_ref, den_ref, xl_ref, cb_ref, pw_ref, h_ref, s_ref):
        for h in range(H):
            hcol = (op_ref[h, 0] + op_ref[h, 1]
                    + exs_ref[:, h:h + 1] * xl_ref[:, h * C:(h + 1) * C]
                    ) / den_ref[:, h:h + 1]
            h_ref[:, pl.ds(h * C, C)] = hcol
        hb = jnp.maximum(h_ref[...] + cb_ref[...], 0.0)
        h_ref[...] = hb
        pw = pw_ref[...]
        nrm = jnp.sqrt(jnp.sum(pw * pw))
        sc = jnp.tanh(jnp.dot(hb, pw, preferred_element_type=jnp.float32) / nrm)
        s_ref[...] = jnp.broadcast_to(sc, (NB, 8))

    return pl.pallas_call(
        body,
        grid=(NBLK,),
        in_specs=[
            pl.BlockSpec((H, NC, NB, C), lambda i: (0, 0, i, 0)),
            pl.BlockSpec((NB, 16), lambda i: (i, 0)),
            pl.BlockSpec((NB, 16), lambda i: (i, 0)),
            pl.BlockSpec((NB, HC), lambda i: (i, 0)),
            pl.BlockSpec((1, HC), lambda i: (0, 0)),
            pl.BlockSpec((HC, 1), lambda i: (0, 0)),
        ],
        out_specs=[
            pl.BlockSpec((NB, HC), lambda i: (i, 0)),
            pl.BlockSpec((NB, 8), lambda i: (i, 0)),
        ],
        out_shape=[
            jax.ShapeDtypeStruct((N, HC), jnp.float32),
            jax.ShapeDtypeStruct((N, 8), jnp.float32),
        ],
    )(out_part, ex_self, denom_pad, x_l, conv_bias2, pool_w2)


def _tc5b_pool(hfull, score, batchf, scoreT, batchT):
    def body(h_ref, s_ref, b_ref, sallT_ref, ballT_ref, out_ref,
             gmp_ref, gap_ref):
        i = pl.program_id(0)

        @pl.when(i == 0)
        def _init():
            gmp_ref[...] = jnp.full((G, HC), -jnp.inf, jnp.float32)
            gap_ref[...] = jnp.zeros((G, HC), jnp.float32)

        s_blk = s_ref[:, 0:1]
        b_blk = b_ref[:, 0:1]
        s_all = sallT_ref[0:1, :]
        b_all = ballT_ref[0:1, :]

        gidx = (i * NB
                + lax.broadcasted_iota(jnp.int32, (NB, 1), 0)).astype(jnp.float32)
        jidx = lax.broadcasted_iota(jnp.int32, (1, N), 1).astype(jnp.float32)
        same = (b_blk == b_all).astype(jnp.float32)
        higher = jnp.where(
            (s_all > s_blk) | ((s_all == s_blk) & (jidx < gidx)), 1.0, 0.0)
        rank = jnp.sum(same * higher, axis=1, keepdims=True)

        cnt = jnp.sum(same, axis=1, keepdims=True)
        kb = jnp.ceil(RATIO * cnt)
        keep = (rank < kb).astype(jnp.float32)

        xp = h_ref[...] * s_blk
        for g in range(G):
            mg = ((b_blk == float(g)).astype(jnp.float32) * keep) > 0.5
            contrib_max = jnp.max(jnp.where(mg, xp, -jnp.inf), axis=0,
                                  keepdims=True)
            contrib_sum = jnp.sum(jnp.where(mg, xp, 0.0), axis=0,
                                  keepdims=True)
            gmp_ref[pl.ds(g, 1), :] = jnp.maximum(gmp_ref[pl.ds(g, 1), :],
                                                  contrib_max)
            gap_ref[pl.ds(g, 1), :] = gap_ref[pl.ds(g, 1), :] + contrib_sum

        @pl.when(i == NBLK - 1)
        def _fin():
            grange = lax.broadcasted_iota(jnp.int32, (G, 1), 0).astype(jnp.float32)
            counts = jnp.sum((b_all == grange).astype(jnp.float32),
                             axis=1, keepdims=True)
            kf = jnp.ceil(RATIO * counts)
            out_ref[:, pl.ds(0, HC)] = gmp_ref[...]
            out_ref[:, pl.ds(HC, HC)] = gap_ref[...] / kf

    return pl.pallas_call(
        body,
        grid=(NBLK,),
        in_specs=[
            pl.BlockSpec((NB, HC), lambda i: (i, 0)),
            pl.BlockSpec((NB, 8), lambda i: (i, 0)),
            pl.BlockSpec((NB, 8), lambda i: (i, 0)),
            pl.BlockSpec((8, N), lambda i: (0, 0)),
            pl.BlockSpec((8, N), lambda i: (0, 0)),
        ],
        out_specs=pl.BlockSpec((G, 2 * HC), lambda i: (0, 0)),
        out_shape=jax.ShapeDtypeStruct((G, 2 * HC), jnp.float32),
        scratch_shapes=[
            pltpu.VMEM((G, HC), jnp.float32),
            pltpu.VMEM((G, HC), jnp.float32),
        ],
    )(hfull, score, batchf, scoreT, batchT)


def kernel(x, edge_index, edge_attr, batch, W_l, b_l, W_r, b_r, W_e, att,
           conv_bias, pool_w):
    src = edge_index[0]
    dst = edge_index[1]
    b_l2 = b_l.reshape(1, HC)
    b_r2 = b_r.reshape(1, HC)
    cb2 = conv_bias.reshape(1, HC)
    pw2 = pool_w.reshape(HC, 1)
    batchf = jnp.broadcast_to(batch.astype(jnp.float32)[:, None], (N, 8))

    ea_pad = jnp.concatenate(
        [edge_attr, jnp.ones((E, 1), jnp.float32),
         jnp.zeros((E, C - D_EDGE - 1), jnp.float32)], axis=1)
    zeros128 = jnp.zeros((N, C), jnp.float32)

    src3g = src.reshape(NW, E // (NW * CHG), CHG)
    dst3g = dst.reshape(NW, E // (NW * CHG), CHG)
    dst3s = dst.reshape(NW, E // (NW * CH), CH)
    zpad = jnp.zeros((NW, E // (NW * CHG), 64 - CHG), jnp.int32)
    idxpack = jnp.concatenate([src3g, zpad, dst3g, zpad], axis=2)

    x_l, x_r = _tc1_proj(x, W_l, b_l2, W_r, b_r2)

    la_part = _sc_scatter_add(ea_pad, dst3s, zeros128, C)
    gxl, gxr = _sc_gather2(x_l, x_r, idxpack, HC)

    ex, msg = _tc2_edge_alpha(gxl, gxr, edge_attr, W_e, att)
    ex_self = _tc2b_self_alpha(x_l, x_r, la_part, W_e, att)

    den_part = _sc_scatter_add(ex, dst3s, zeros128, C)
    denom_pad = _tc3_denom(den_part, ex_self)

    out_part = _sc_scatter_msg(msg, dst3s, zeros128)

    hfull, score = _tc5a_combine(out_part, ex_self, denom_pad, x_l, cb2, pw2)
    scoreT = jnp.broadcast_to(score[:, 0].reshape(1, N), (8, N))
    batchT = jnp.broadcast_to(batch.astype(jnp.float32).reshape(1, N), (8, N))
    return _tc5b_pool(hfull, score, batchf, scoreT, batchT)
